# baseline (speedup 1.0000x reference)
import functools

import jax
import jax.numpy as jnp
from jax import lax
from jax.experimental import pallas as pl
from jax.experimental.pallas import tpu as pltpu
from jax.experimental.pallas import tpu_sc as plsc

NUM_NODES = 1000000
NUM_HIGH = 100000
NUM_LOW = NUM_NODES - NUM_HIGH
D_HIGH = 128
D_LOW = 32
D_COMMON = 64
B = 16384

LOW_PER_BLK = D_HIGH // D_LOW
NUM_LOW_BLK = NUM_LOW // LOW_PER_BLK

NC = 2
NS = 16
NW = NC * NS
B_PER_W = B // NW
IDX_CHUNK = 128
N_CHUNKS = B_PER_W // IDX_CHUNK
L = 16
IDX_CHUNK_L = IDX_CHUNK // L
HALF = B_PER_W // 2


def _sc_gather(node_ids, emb_high, emb_low_blk):
    mesh = plsc.VectorSubcoreMesh(
        core_axis_name="c", subcore_axis_name="s", num_cores=NC, num_subcores=NS
    )

    @functools.partial(
        pl.kernel,
        out_type=(
            jax.ShapeDtypeStruct((B, D_HIGH), jnp.float32),
            jax.ShapeDtypeStruct((B, D_HIGH), jnp.float32),
            jax.ShapeDtypeStruct((B, L), jnp.float32),
        ),
        mesh=mesh,
        compiler_params=pltpu.CompilerParams(needs_layout_passes=False),
        scratch_types=[
            pltpu.VMEM((B_PER_W,), jnp.int32),
            pltpu.VMEM((N_CHUNKS, IDX_CHUNK), jnp.int32),
            pltpu.VMEM((N_CHUNKS, IDX_CHUNK), jnp.int32),
            pltpu.VMEM((HALF, D_HIGH), jnp.float32),
            pltpu.VMEM((IDX_CHUNK, D_HIGH), jnp.float32),
            pltpu.VMEM((B_PER_W, L), jnp.float32),
            pltpu.SemaphoreType.DMA,
            pltpu.SemaphoreType.DMA,
        ],
    )
    def k(ids_hbm, eh_hbm, el_hbm, gh_hbm, gl_hbm, ind_hbm,
          ids_v, hidx_v, lidx_v, hbuf, lbuf, indbuf, sem_h, sem_l):
        wid = lax.axis_index("s") * NC + lax.axis_index("c")
        base = wid * B_PER_W
        pltpu.sync_copy(ids_hbm.at[pl.ds(base, B_PER_W)], ids_v)
        iota = lax.iota(jnp.int32, L)
        zero = jnp.zeros((L,), jnp.float32)
        one = jnp.ones((L,), jnp.float32)
        for c in range(B_PER_W // L):
            ids = ids_v[pl.ds(c * L, L)]
            p = base + c * L + iota
            is_h = ids < NUM_HIGH
            lowraw = jnp.clip(ids - NUM_HIGH, 0, NUM_LOW - 1)
            hidx_v[c // IDX_CHUNK_L, pl.ds((c % IDX_CHUNK_L) * L, L)] = (
                jnp.where(is_h, ids, p))
            lidx = ((lowraw >> 12) << 10) | (lowraw & 1023)
            lidx_v[c // IDX_CHUNK_L, pl.ds((c % IDX_CHUNK_L) * L, L)] = (
                jnp.where(is_h, p, lidx))
            rem = (lowraw >> 10) & 3
            mh = jnp.where(is_h, one, zero)
            ml = one - mh
            rows = c * L + iota
            for r in range(LOW_PER_BLK):
                plsc.store_scatter(
                    indbuf, [rows, jnp.full((L,), r, jnp.int32)],
                    jnp.where(rem == r, ml, zero))
            plsc.store_scatter(indbuf, [rows, jnp.full((L,), 4, jnp.int32)],
                               mh)
            plsc.store_scatter(indbuf, [rows, jnp.full((L,), 5, jnp.int32)],
                               ml)

        def high_round(r):
            return [
                pltpu.async_copy(
                    eh_hbm.at[hidx_v.at[2 * r + j]],
                    hbuf.at[pl.ds(j * IDX_CHUNK, IDX_CHUNK)], sem_h)
                for j in range(2)
            ]

        def low_round(r):
            return pltpu.async_copy(el_hbm.at[lidx_v.at[r]], lbuf, sem_l)

        lc = low_round(0)
        hc = high_round(0)
        lc.wait()
        pltpu.sync_copy(lbuf, gl_hbm.at[pl.ds(base, IDX_CHUNK)])
        lc = low_round(1)
        for c0 in hc:
            c0.wait()
        pltpu.sync_copy(hbuf, gh_hbm.at[pl.ds(base, HALF)])
        hc = high_round(1)
        for r in (1, 2):
            lc.wait()
            pltpu.sync_copy(
                lbuf, gl_hbm.at[pl.ds(base + r * IDX_CHUNK, IDX_CHUNK)])
            lc = low_round(r + 1)
        for c0 in hc:
            c0.wait()
        pltpu.sync_copy(hbuf, gh_hbm.at[pl.ds(base + HALF, HALF)])
        lc.wait()
        pltpu.sync_copy(
            lbuf, gl_hbm.at[pl.ds(base + 3 * IDX_CHUNK, IDX_CHUNK)])
        pltpu.sync_copy(indbuf, ind_hbm.at[pl.ds(base, B_PER_W)])

    return k(node_ids, emb_high, emb_low_blk)


RP_C = 4096
RP_GRID = -(-NUM_LOW // RP_C)


RP_V = RP_C // LOW_PER_BLK


def _repack_body(x_ref, i_ref, o_ref):
    for j in range(LOW_PER_BLK):
        o_ref[:, 32 * j:32 * (j + 1)] = lax.dot_general(
            x_ref[:, j * RP_V:(j + 1) * RP_V], i_ref[...],
            (((0,), (0,)), ((), ())), preferred_element_type=jnp.float32)


def _repack_low(emb_low):
    return pl.pallas_call(
        _repack_body,
        grid=(RP_GRID,),
        in_specs=[pl.BlockSpec((D_LOW, RP_C), lambda i: (0, i)),
                  pl.BlockSpec((D_LOW, D_LOW), lambda i: (0, 0))],
        out_specs=pl.BlockSpec((RP_C // LOW_PER_BLK, D_HIGH), lambda i: (i, 0)),
        out_shape=jax.ShapeDtypeStruct((RP_GRID * RP_V, D_HIGH), jnp.float32),
    )(emb_low.T, jnp.eye(D_LOW, dtype=jnp.float32))


BLK = 4096


def _tc_body(gh_ref, gl_ref, ind_ref, wh_ref, scat_ref, bh_ref, bl_ref,
             out_ref):
    h = lax.dot_general(gh_ref[...], wh_ref[...],
                        (((1,), (1,)), ((), ())),
                        preferred_element_type=jnp.float32)
    l4 = lax.dot_general(gl_ref[...], scat_ref[...],
                         (((1,), (0,)), ((), ())),
                         preferred_element_type=jnp.float32)
    ind = ind_ref[...]
    acc = h * ind[:, 4:5]
    for r in range(LOW_PER_BLK):
        acc = acc + l4[:, r * D_COMMON:(r + 1) * D_COMMON] * ind[:, r:r + 1]
    out_ref[...] = acc + ind[:, 4:5] * bh_ref[...] + ind[:, 5:6] * bl_ref[...]


def _tc_project(gh, gl, ind, W_high, b_high, W_low, b_low):
    scat = jnp.kron(jnp.eye(LOW_PER_BLK, dtype=jnp.float32), W_low.T)
    return pl.pallas_call(
        _tc_body,
        grid=(B // BLK,),
        in_specs=[
            pl.BlockSpec((BLK, D_HIGH), lambda i: (i, 0)),
            pl.BlockSpec((BLK, D_HIGH), lambda i: (i, 0)),
            pl.BlockSpec((BLK, L), lambda i: (i, 0)),
            pl.BlockSpec((D_COMMON, D_HIGH), lambda i: (0, 0)),
            pl.BlockSpec((D_HIGH, LOW_PER_BLK * D_COMMON), lambda i: (0, 0)),
            pl.BlockSpec((1, D_COMMON), lambda i: (0, 0)),
            pl.BlockSpec((1, D_COMMON), lambda i: (0, 0)),
        ],
        out_specs=pl.BlockSpec((BLK, D_COMMON), lambda i: (i, 0)),
        out_shape=jax.ShapeDtypeStruct((B, D_COMMON), jnp.float32),
    )(gh, gl, ind, W_high, scat,
      b_high.reshape(1, D_COMMON), b_low.reshape(1, D_COMMON))


def kernel(node_ids, emb_high, emb_low, W_high, b_high, W_low, b_low):
    emb_low_blk = _repack_low(emb_low)
    gh, gl, ind = _sc_gather(node_ids, emb_high, emb_low_blk)
    return _tc_project(gh, gl, ind, W_high, b_high, W_low, b_low)


# speedup vs baseline: 1.1356x; 1.1356x over previous
import functools

import jax
import jax.numpy as jnp
from jax import lax
from jax.experimental import pallas as pl
from jax.experimental.pallas import tpu as pltpu
from jax.experimental.pallas import tpu_sc as plsc

NUM_NODES = 1000000
NUM_HIGH = 100000
NUM_LOW = NUM_NODES - NUM_HIGH
D_HIGH = 128
D_LOW = 32
D_COMMON = 64
B = 16384

LOW_PER_BLK = D_HIGH // D_LOW
NUM_LOW_BLK = NUM_LOW // LOW_PER_BLK

NC = 2
NS = 16
NW = NC * NS
B_PER_W = B // NW
IDX_CHUNK = 128
N_CHUNKS = B_PER_W // IDX_CHUNK
L = 16
IDX_CHUNK_L = IDX_CHUNK // L
HALF = B_PER_W // 2
RP_C = 8192
RP_V = RP_C // 4
RP_C_LOG = 13
RP_V_LOG = 11


def _sc_gather(node_ids, emb_high, emb_low_blk):
    mesh = plsc.VectorSubcoreMesh(
        core_axis_name="c", subcore_axis_name="s", num_cores=NC, num_subcores=NS
    )

    @functools.partial(
        pl.kernel,
        out_type=(
            jax.ShapeDtypeStruct((B, D_HIGH), jnp.float32),
            jax.ShapeDtypeStruct((B, D_HIGH), jnp.float32),
            jax.ShapeDtypeStruct((B, L), jnp.float32),
        ),
        mesh=mesh,
        compiler_params=pltpu.CompilerParams(needs_layout_passes=False),
        scratch_types=[
            pltpu.VMEM((B_PER_W,), jnp.int32),
            pltpu.VMEM((N_CHUNKS, IDX_CHUNK), jnp.int32),
            pltpu.VMEM((N_CHUNKS, IDX_CHUNK), jnp.int32),
            pltpu.VMEM((HALF, D_HIGH), jnp.float32),
            pltpu.VMEM((IDX_CHUNK, D_HIGH), jnp.float32),
            pltpu.VMEM((B_PER_W, L), jnp.float32),
            pltpu.SemaphoreType.DMA,
            pltpu.SemaphoreType.DMA,
        ],
    )
    def k(ids_hbm, eh_hbm, el_hbm, gh_hbm, gl_hbm, ind_hbm,
          ids_v, hidx_v, lidx_v, hbuf, lbuf, indbuf, sem_h, sem_l):
        wid = lax.axis_index("s") * NC + lax.axis_index("c")
        base = wid * B_PER_W
        pltpu.sync_copy(ids_hbm.at[pl.ds(base, B_PER_W)], ids_v)
        iota = lax.iota(jnp.int32, L)
        zero = jnp.zeros((L,), jnp.float32)
        one = jnp.ones((L,), jnp.float32)
        for c in range(B_PER_W // L):
            ids = ids_v[pl.ds(c * L, L)]
            p = base + c * L + iota
            is_h = ids < NUM_HIGH
            lowraw = jnp.clip(ids - NUM_HIGH, 0, NUM_LOW - 1)
            hidx_v[c // IDX_CHUNK_L, pl.ds((c % IDX_CHUNK_L) * L, L)] = (
                jnp.where(is_h, ids, p))
            lidx = ((lowraw >> RP_C_LOG) << RP_V_LOG) | (lowraw & (RP_V - 1))
            lidx_v[c // IDX_CHUNK_L, pl.ds((c % IDX_CHUNK_L) * L, L)] = (
                jnp.where(is_h, p, lidx))
            rem = (lowraw >> RP_V_LOG) & (LOW_PER_BLK - 1)
            mh = jnp.where(is_h, one, zero)
            ml = one - mh
            rows = c * L + iota
            for r in range(LOW_PER_BLK):
                plsc.store_scatter(
                    indbuf, [rows, jnp.full((L,), r, jnp.int32)],
                    jnp.where(rem == r, ml, zero))
            plsc.store_scatter(indbuf, [rows, jnp.full((L,), 4, jnp.int32)],
                               mh)
            plsc.store_scatter(indbuf, [rows, jnp.full((L,), 5, jnp.int32)],
                               ml)

        def high_round(r):
            return [
                pltpu.async_copy(
                    eh_hbm.at[hidx_v.at[2 * r + j]],
                    hbuf.at[pl.ds(j * IDX_CHUNK, IDX_CHUNK)], sem_h)
                for j in range(2)
            ]

        def low_round(r):
            return pltpu.async_copy(el_hbm.at[lidx_v.at[r]], lbuf, sem_l)

        lc = low_round(0)
        hc = high_round(0)
        lc.wait()
        pltpu.sync_copy(lbuf, gl_hbm.at[pl.ds(base, IDX_CHUNK)])
        lc = low_round(1)
        for c0 in hc:
            c0.wait()
        pltpu.sync_copy(hbuf, gh_hbm.at[pl.ds(base, HALF)])
        hc = high_round(1)
        for r in (1, 2):
            lc.wait()
            pltpu.sync_copy(
                lbuf, gl_hbm.at[pl.ds(base + r * IDX_CHUNK, IDX_CHUNK)])
            lc = low_round(r + 1)
        for c0 in hc:
            c0.wait()
        pltpu.sync_copy(hbuf, gh_hbm.at[pl.ds(base + HALF, HALF)])
        lc.wait()
        pltpu.sync_copy(
            lbuf, gl_hbm.at[pl.ds(base + 3 * IDX_CHUNK, IDX_CHUNK)])
        pltpu.sync_copy(indbuf, ind_hbm.at[pl.ds(base, B_PER_W)])

    return k(node_ids, emb_high, emb_low_blk)


RP_GRID = -(-NUM_LOW // RP_C)


def _repack_body(x_ref, i_ref, o_ref):
    for j in range(LOW_PER_BLK):
        o_ref[:, 32 * j:32 * (j + 1)] = lax.dot_general(
            x_ref[:, j * RP_V:(j + 1) * RP_V], i_ref[...],
            (((0,), (0,)), ((), ())), preferred_element_type=jnp.float32)


def _repack_low(emb_low):
    return pl.pallas_call(
        _repack_body,
        grid=(RP_GRID,),
        in_specs=[pl.BlockSpec((D_LOW, RP_C), lambda i: (0, i)),
                  pl.BlockSpec((D_LOW, D_LOW), lambda i: (0, 0))],
        out_specs=pl.BlockSpec((RP_C // LOW_PER_BLK, D_HIGH), lambda i: (i, 0)),
        out_shape=jax.ShapeDtypeStruct((RP_GRID * RP_V, D_HIGH), jnp.float32),
    )(emb_low.T, jnp.eye(D_LOW, dtype=jnp.float32))


BLK = 4096


def _tc_body(gh_ref, gl_ref, ind_ref, wh_ref, scat_ref, bh_ref, bl_ref,
             out_ref):
    h = lax.dot_general(gh_ref[...], wh_ref[...],
                        (((1,), (1,)), ((), ())),
                        preferred_element_type=jnp.float32)
    l4 = lax.dot_general(gl_ref[...], scat_ref[...],
                         (((1,), (0,)), ((), ())),
                         preferred_element_type=jnp.float32)
    ind = ind_ref[...]
    acc = h * ind[:, 4:5]
    for r in range(LOW_PER_BLK):
        acc = acc + l4[:, r * D_COMMON:(r + 1) * D_COMMON] * ind[:, r:r + 1]
    out_ref[...] = acc + ind[:, 4:5] * bh_ref[...] + ind[:, 5:6] * bl_ref[...]


def _tc_project(gh, gl, ind, W_high, b_high, W_low, b_low):
    scat = jnp.kron(jnp.eye(LOW_PER_BLK, dtype=jnp.float32), W_low.T)
    return pl.pallas_call(
        _tc_body,
        grid=(B // BLK,),
        in_specs=[
            pl.BlockSpec((BLK, D_HIGH), lambda i: (i, 0)),
            pl.BlockSpec((BLK, D_HIGH), lambda i: (i, 0)),
            pl.BlockSpec((BLK, L), lambda i: (i, 0)),
            pl.BlockSpec((D_COMMON, D_HIGH), lambda i: (0, 0)),
            pl.BlockSpec((D_HIGH, LOW_PER_BLK * D_COMMON), lambda i: (0, 0)),
            pl.BlockSpec((1, D_COMMON), lambda i: (0, 0)),
            pl.BlockSpec((1, D_COMMON), lambda i: (0, 0)),
        ],
        out_specs=pl.BlockSpec((BLK, D_COMMON), lambda i: (i, 0)),
        out_shape=jax.ShapeDtypeStruct((B, D_COMMON), jnp.float32),
    )(gh, gl, ind, W_high, scat,
      b_high.reshape(1, D_COMMON), b_low.reshape(1, D_COMMON))


def kernel(node_ids, emb_high, emb_low, W_high, b_high, W_low, b_low):
    emb_low_blk = _repack_low(emb_low)
    gh, gl, ind = _sc_gather(node_ids, emb_high, emb_low_blk)
    return _tc_project(gh, gl, ind, W_high, b_high, W_low, b_low)
